# baseline (speedup 1.0000x reference)
import functools

import jax
import jax.numpy as jnp
from jax import lax
from jax.experimental import pallas as pl
from jax.experimental.pallas import tpu as pltpu
from jax.experimental.pallas import tpu_sc as plsc

_B, _C, _M, _N, _K = 8, 256, 1024, 2048, 3
_C3 = 3 * _C
_HID = (3 * _C) // 2
_OUT = 128
_BN = _B * _N


_TA = 384
_HPAD = 512


def _a2_body(y_ref, r_ref, w10_ref, w11_ref, w12_ref, vx_ref,
             proj_ref, vc_ref, ft_ref, a0_ref):
    iu = lax.broadcasted_iota(jnp.int32, (_TA, _TA), 0)
    iv = lax.broadcasted_iota(jnp.int32, (_TA, _TA), 1)
    sameq = (iu // 3) == (iv // 3)
    um, vm = iu % 3, iv % 3
    acc = None
    for i in range(3):
        for j in range(3):
            t = jnp.where(sameq & (um == i) & (vm == j),
                          r_ref[0, i:i + 1, j:j + 1], 0.0)
            acc = t if acc is None else acc + t
    a0_ref[...] = acc
    for t in range(3 * _M // _TA):
        lo, hi = t * _TA, (t + 1) * _TA
        ft_ref[:, lo:hi] = jnp.dot(
            y_ref[0, :, lo:hi].astype(jnp.bfloat16),
            a0_ref[...].astype(jnp.bfloat16),
            preferred_element_type=jnp.float32)
    w1s = (w10_ref, w11_ref, w12_ref)
    acc = None
    for k in range(3):
        pk = lax.dot_general(
            ft_ref[:, k * _M:(k + 1) * _M].astype(jnp.bfloat16),
            w1s[k][...].astype(jnp.bfloat16),
            (((0,), (0,)), ((), ())),
            preferred_element_type=jnp.float32)
        acc = pk if acc is None else acc + pk
    proj_ref[0, :, 0:_HID] = acc.astype(jnp.bfloat16)
    proj_ref[0, :, _HID:] = jnp.zeros((_M, _HPAD - _HID), jnp.bfloat16)
    def _bf(x):
        return x.astype(jnp.bfloat16).astype(jnp.float32)

    for j in range(3):
        vc_ref[0, j:j + 1, :] = (
            _bf(r_ref[0, 0:1, j:j + 1]) * _bf(vx_ref[0, 0:1, :])
            + _bf(r_ref[0, 1:2, j:j + 1]) * _bf(vx_ref[0, 1:2, :])
            + _bf(r_ref[0, 2:3, j:j + 1]) * _bf(vx_ref[0, 2:3, :]))


def _stage_a2(y, r_align, w10, w11, w12, vn_xyz):
    return pl.pallas_call(
        _a2_body,
        grid=(_B,),
        in_specs=[
            pl.BlockSpec((1, _C, 3 * _M), lambda b: (b, 0, 0)),
            pl.BlockSpec((1, 3, 3), lambda b: (b, 0, 0)),
            pl.BlockSpec((_C, _HID), lambda b: (0, 0)),
            pl.BlockSpec((_C, _HID), lambda b: (0, 0)),
            pl.BlockSpec((_C, _HID), lambda b: (0, 0)),
            pl.BlockSpec((1, 3, _M), lambda b: (b, 0, 0)),
        ],
        out_specs=[
            pl.BlockSpec((1, _M, _HPAD), lambda b: (b, 0, 0)),
            pl.BlockSpec((1, 3, _M), lambda b: (b, 0, 0)),
        ],
        out_shape=[
            jax.ShapeDtypeStruct((_B, _M, _HPAD), jnp.bfloat16),
            jax.ShapeDtypeStruct((_B, 3, _M), jnp.float32),
        ],
        scratch_shapes=[pltpu.VMEM((_C, 3 * _M), jnp.float32),
                        pltpu.VMEM((_TA, _TA), jnp.float32)],
    )(y, r_align, w10, w11, w12, vn_xyz)


_TN = 256


def _stage_b_body(tt_ref, vc_ref, idx_ref, w_ref):
    b = pl.program_id(0)
    t = tt_ref[0]
    v = vc_ref[0]
    d2 = None
    for i in range(3):
        diff = t[:, i:i + 1] - v[i:i + 1, :]
        sq = diff * diff
        d2 = sq if d2 is None else d2 + sq
    iota = lax.broadcasted_iota(jnp.int32, (_TN, _M), 1)
    inf = jnp.float32(jnp.inf)
    cur = d2
    vals, idxs = [], []
    for k in range(_K):
        mv = jnp.min(cur, axis=1, keepdims=True)
        mi = jnp.min(jnp.where(cur == mv, iota, _M), axis=1,
                     keepdims=True)
        vals.append(mv)
        idxs.append(mi)
        if k < _K - 1:
            cur = jnp.where(iota == mi, inf, cur)
    inv = [1.0 / (jnp.sqrt(jnp.maximum(vv, 0.0)) + 1e-8) for vv in vals]
    s = inv[0] + inv[1] + inv[2]
    w_ref[0] = jnp.concatenate(
        [jnp.broadcast_to((ik / s).astype(jnp.bfloat16), (_TN, 2 * _L))
         for ik in inv], axis=1)
    idx_ref[0] = jnp.concatenate(idxs, axis=1) + b * _M


def _stage_b(target_t, vc):
    return pl.pallas_call(
        _stage_b_body,
        grid=(_B, _N // _TN),
        in_specs=[
            pl.BlockSpec((1, _TN, 3), lambda b, n: (b, n, 0)),
            pl.BlockSpec((1, 3, _M), lambda b, n: (b, 0, 0)),
        ],
        out_specs=[
            pl.BlockSpec((1, _TN, 3), lambda b, n: (b, n, 0)),
            pl.BlockSpec((1, _TN, 6 * _L), lambda b, n: (b, n, 0)),
        ],
        out_shape=[
            jax.ShapeDtypeStruct((_B, _N, 3), jnp.int32),
            jax.ShapeDtypeStruct((_B, _N, 6 * _L), jnp.bfloat16),
        ],
    )(target_t, vc)


_NC, _NS, _L = 2, 16, 16
_NW = _NC * _NS
_TPW = _BN // _NW
_CH = 16
_NCH = _TPW // _CH


def _stage_c_body(proj_hbm, idx_hbm, w_hbm, h_hbm, idx_v, w_v,
                  rows0, rows1, out_v, g0, g1, o0, o1):
    wid = lax.axis_index("s") * _NC + lax.axis_index("c")
    base = wid * _TPW
    rows = (rows0, rows1)
    gsem = (g0, g1)
    osem = (o0, o1)

    pltpu.sync_copy(idx_hbm.at[pl.ds(base * 3, _TPW * 3)], idx_v)
    pltpu.sync_copy(w_hbm.at[pl.ds(base * 3 * _L, _TPW * 3 * _L)], w_v)

    def gather_start(ch, par):
        pltpu.async_copy(
            proj_hbm.at[idx_v.at[pl.ds(ch * (_CH * 3), _CH * 3)]],
            rows[par], gsem[par])

    def gather_wait(par):
        pltpu.make_async_copy(
            proj_hbm.at[idx_v.at[pl.ds(0, _CH * 3)]],
            rows[par], gsem[par]).wait()

    def out_start(ch, par):
        pltpu.async_copy(out_v.at[par],
                         h_hbm.at[pl.ds(base + ch * _CH, _CH)], osem[par])

    def out_wait(par):
        pltpu.make_async_copy(out_v.at[par],
                              h_hbm.at[pl.ds(base, _CH)], osem[par]).wait()

    def compute(ch, par):
        rv = rows[par]

        def tgt(t, carry):
            wb = ch * (_CH * 3 * _L) + t * (3 * _L)
            w0 = plsc.bitcast(w_v[pl.ds(wb, _L)], jnp.bfloat16)
            w1 = plsc.bitcast(w_v[pl.ds(wb + _L, _L)], jnp.bfloat16)
            w2 = plsc.bitcast(w_v[pl.ds(wb + 2 * _L, _L)], jnp.bfloat16)
            for d in range(_HID // (2 * _L)):
                sl = pl.ds(d * _L, _L)
                r0 = plsc.bitcast(rv[3 * t, sl], jnp.bfloat16)
                r1 = plsc.bitcast(rv[3 * t + 1, sl], jnp.bfloat16)
                r2 = plsc.bitcast(rv[3 * t + 2, sl], jnp.bfloat16)
                out_v[par, t, sl] = plsc.bitcast(
                    r0 * w0 + r1 * w1 + r2 * w2, jnp.float32)
            return carry

        lax.fori_loop(0, _CH, tgt, 0)

    gather_start(0, 0)
    gather_start(1, 1)
    for par in (0, 1):
        gather_wait(par)
        compute(par, par)
        out_start(par, par)
        gather_start(par + 2, par)

    def pair(ph, carry):
        for par in (0, 1):
            ch = 2 * ph + par
            gather_wait(par)
            out_wait(par)
            compute(ch, par)
            out_start(ch, par)
            gather_start(ch + 2, par)
        return carry

    lax.fori_loop(1, _NCH // 2 - 1, pair, 0)

    for par in (0, 1):
        ch = _NCH - 2 + par
        gather_wait(par)
        out_wait(par)
        compute(ch, par)
        out_start(ch, par)
    for par in (0, 1):
        out_wait(par)


@functools.cache
def _make_stage_c():
    return pl.kernel(
        _stage_c_body,
        out_type=jax.ShapeDtypeStruct((_BN, _HID // 2), jnp.float32),
        mesh=plsc.VectorSubcoreMesh(core_axis_name="c", subcore_axis_name="s"),
        compiler_params=pltpu.CompilerParams(needs_layout_passes=False),
        scratch_types=[
            pltpu.VMEM((_TPW * 3,), jnp.int32),
            pltpu.VMEM((_TPW * 3 * _L,), jnp.float32),
            pltpu.VMEM((_CH * 3, _HPAD // 2), jnp.float32),
            pltpu.VMEM((_CH * 3, _HPAD // 2), jnp.float32),
            pltpu.VMEM((2, _CH, _HID // 2), jnp.float32),
            pltpu.SemaphoreType.DMA,
            pltpu.SemaphoreType.DMA,
            pltpu.SemaphoreType.DMA,
            pltpu.SemaphoreType.DMA,
        ],
    )


_TD = 2048


def _stage_d_body(h_ref, b1_ref, w2_ref, b2_ref, o_ref):
    x = jnp.maximum(h_ref[0].astype(jnp.float32) + b1_ref[...], 0.0)
    o_ref[0] = (lax.dot_general(w2_ref[...], x.astype(jnp.bfloat16),
                                (((0,), (1,)), ((), ())),
                                preferred_element_type=jnp.float32)
                + b2_ref[...])


def _stage_d(h, b1, w2, b2t):
    return pl.pallas_call(
        _stage_d_body,
        grid=(_B, _N // _TD),
        in_specs=[
            pl.BlockSpec((1, _TD, _HID), lambda b, n: (b, n, 0)),
            pl.BlockSpec((1, _HID), lambda b, n: (0, 0)),
            pl.BlockSpec((_HID, _OUT), lambda b, n: (0, 0)),
            pl.BlockSpec((_OUT, 1), lambda b, n: (0, 0)),
        ],
        out_specs=pl.BlockSpec((1, _OUT, _TD), lambda b, n: (b, 0, n)),
        out_shape=jax.ShapeDtypeStruct((_B, _OUT, _N), jnp.float32),
    )(h, b1, w2, b2t)


def kernel(vn_feat, vn_xyz, target_xyz, R_align, W1, b1, W2, b2):
    y = vn_feat.reshape(_B, _C, 3 * _M)
    w13 = W1.reshape(_C, 3, _HID)
    w10, w11, w12 = w13[:, 0, :], w13[:, 1, :], w13[:, 2, :]
    target_t = target_xyz.transpose(0, 2, 1)

    proj, vc = _stage_a2(y, R_align, w10, w11, w12, vn_xyz)
    idxg, w = _stage_b(target_t, vc)

    projp = lax.bitcast_convert_type(
        proj.reshape(_B * _M, _HPAD // 2, 2), jnp.float32)
    wp = lax.bitcast_convert_type(
        w.reshape(_BN * 3 * _L, 2), jnp.float32)
    h = _make_stage_c()(projp, idxg.reshape(_BN * 3), wp)
    hb = lax.bitcast_convert_type(h, jnp.bfloat16).reshape(_B, _N, _HID)

    return _stage_d(hb, b1.reshape(1, _HID),
                    W2.astype(jnp.bfloat16), b2.reshape(_OUT, 1))
